# baseline (speedup 1.0000x reference)
import functools

import jax
import jax.numpy as jnp
from jax.experimental import pallas as pl
from jax.experimental.pallas import tpu as pltpu

D_MODEL = 4096
N_EXPERTS = 64
K = 8

BT = 1024
NK = 2
DK = D_MODEL // NK

NEG_INF = float("-inf")


def _router_body(x_ref, wt_ref, w_ref, i_ref, p_ref, acc_ref):
    k = pl.program_id(1)

    part = jnp.dot(x_ref[...], wt_ref[...],
                   preferred_element_type=jnp.float32)

    @pl.when(k == 0)
    def _init():
        acc_ref[...] = part

    @pl.when(k != 0)
    def _acc():
        acc_ref[...] += part

    @pl.when(k == NK - 1)
    def _epilogue():
        logits = acc_ref[...]

        iota = jax.lax.broadcasted_iota(jnp.int32, logits.shape, 1)
        work = logits
        vals = []
        idxs = []
        for _ in range(K):
            mx = jnp.max(work, axis=-1, keepdims=True)
            hit = work == mx
            idx = jnp.min(jnp.where(hit, iota, N_EXPERTS),
                          axis=-1, keepdims=True)
            vals.append(mx)
            idxs.append(idx)
            work = jnp.where(iota == idx, NEG_INF, work)

        topv = jnp.concatenate(vals, axis=1)
        topi = jnp.concatenate(idxs, axis=1)

        gex = jnp.exp(topv - topv[:, :1])
        w_ref[...] = gex / jnp.sum(gex, axis=-1, keepdims=True)
        i_ref[...] = topi

        ex = jnp.exp(logits - vals[0])
        p_ref[...] = ex / jnp.sum(ex, axis=-1, keepdims=True)


@jax.jit
def kernel(x, W):
    B, S, D = x.shape
    T = B * S
    xf = x.reshape(T, D)
    wt = W.T

    grid = (T // BT, NK)
    weights, indices, probs = pl.pallas_call(
        _router_body,
        grid=grid,
        in_specs=[
            pl.BlockSpec((BT, DK), lambda i, k: (i, k)),
            pl.BlockSpec((DK, N_EXPERTS), lambda i, k: (k, 0)),
        ],
        out_specs=[
            pl.BlockSpec((BT, K), lambda i, k: (i, 0)),
            pl.BlockSpec((BT, K), lambda i, k: (i, 0)),
            pl.BlockSpec((BT, N_EXPERTS), lambda i, k: (i, 0)),
        ],
        out_shape=[
            jax.ShapeDtypeStruct((T, K), jnp.float32),
            jax.ShapeDtypeStruct((T, K), jnp.int32),
            jax.ShapeDtypeStruct((T, N_EXPERTS), jnp.float32),
        ],
        scratch_shapes=[pltpu.VMEM((BT, N_EXPERTS), jnp.float32)],
        compiler_params=pltpu.CompilerParams(
            dimension_semantics=("arbitrary", "arbitrary"),
        ),
    )(xf, wt)

    return (weights.reshape(B, S, K),
            indices.reshape(B, S, K),
            probs.reshape(B, S, N_EXPERTS))


# speedup vs baseline: 1.2119x; 1.2119x over previous
import functools

import jax
import jax.numpy as jnp
from jax.experimental import pallas as pl
from jax.experimental.pallas import tpu as pltpu

D_MODEL = 4096
N_EXPERTS = 64
K = 8

BT = 1024

NEG_INF = float("-inf")


def _router_body(x_ref, w_ref, wout_ref, i_ref, p_ref):
    logits = jax.lax.dot_general(
        x_ref[...], w_ref[...],
        dimension_numbers=(((1,), (1,)), ((), ())),
        preferred_element_type=jnp.float32)

    iota = jax.lax.broadcasted_iota(jnp.int32, logits.shape, 1)
    work = logits
    vals = []
    idxs = []
    for _ in range(K):
        mx = jnp.max(work, axis=-1, keepdims=True)
        hit = work == mx
        idx = jnp.min(jnp.where(hit, iota, N_EXPERTS), axis=-1, keepdims=True)
        vals.append(mx)
        idxs.append(idx)
        work = jnp.where(iota == idx, NEG_INF, work)

    topv = jnp.concatenate(vals, axis=1)
    topi = jnp.concatenate(idxs, axis=1)

    gex = jnp.exp(topv - topv[:, :1])
    wout_ref[...] = gex / jnp.sum(gex, axis=-1, keepdims=True)
    i_ref[...] = topi

    ex = jnp.exp(logits - vals[0])
    p_ref[...] = ex / jnp.sum(ex, axis=-1, keepdims=True)


@jax.jit
def kernel(x, W):
    B, S, D = x.shape
    T = B * S
    xf = x.reshape(T, D)

    grid = (T // BT,)
    weights, indices, probs = pl.pallas_call(
        _router_body,
        grid=grid,
        in_specs=[
            pl.BlockSpec((BT, D), lambda i: (i, 0)),
            pl.BlockSpec((N_EXPERTS, D), lambda i: (0, 0)),
        ],
        out_specs=[
            pl.BlockSpec((BT, K), lambda i: (i, 0)),
            pl.BlockSpec((BT, K), lambda i: (i, 0)),
            pl.BlockSpec((BT, N_EXPERTS), lambda i: (i, 0)),
        ],
        out_shape=[
            jax.ShapeDtypeStruct((T, K), jnp.float32),
            jax.ShapeDtypeStruct((T, K), jnp.int32),
            jax.ShapeDtypeStruct((T, N_EXPERTS), jnp.float32),
        ],
        compiler_params=pltpu.CompilerParams(
            dimension_semantics=("arbitrary",),
        ),
    )(xf, W)

    return (weights.reshape(B, S, K),
            indices.reshape(B, S, K),
            probs.reshape(B, S, N_EXPERTS))
